# baseline (speedup 1.0000x reference)
import math

import jax
import jax.numpy as jnp
from jax.experimental import pallas as pl
from jax.experimental.pallas import tpu as pltpu

_B, _T, _D, _K = 16, 512, 128, 64
_H = _T // 2


def _hmm_kernel(seqc_ref, len_ref, pbig_ref, pxt_ref, py_ref, out_ref,
                ecat_ref):
    Pbig = pbig_ref[...]
    PT = pxt_ref[...]
    py = py_ref[...]
    log_py = jnp.log(py)
    log_1mpy = jnp.log1p(-py)
    log_px = jnp.log(Pbig[:_K, :_K])

    W = (log_py - log_1mpy).T
    zW = jnp.zeros_like(W)
    Wbig = jnp.concatenate(
        [jnp.concatenate([W, zW], axis=1),
         jnp.concatenate([zW, W], axis=1)], axis=0)
    bias = jnp.sum(log_1mpy, axis=1)[None, :]
    bias2 = jnp.concatenate([bias, bias], axis=1)
    seqc = seqc_ref[...].reshape(_H * _B, 2 * _D)
    emis = jnp.dot(seqc, Wbig, preferred_element_type=jnp.float32) + bias2

    lane = jax.lax.broadcasted_iota(jnp.int32, (_H * _B, 2 * _K), 1)
    is_f = lane < _K
    neg = jnp.float32(-1e30)
    mf = jnp.max(jnp.where(is_f, emis, neg), axis=1, keepdims=True)
    mb = jnp.max(jnp.where(is_f, neg, emis), axis=1, keepdims=True)
    mbrd = jnp.where(is_f, mf, mb)

    len_b2k = len_ref[...]
    i2 = jax.lax.broadcasted_iota(jnp.int32, (_H, _B), 0)
    len_row = len_b2k[:, :1].reshape(1, _B)
    maskf_2d = i2 < len_row
    maskb_2d = (_T - 1 - i2) < len_row
    m_sum = (jnp.sum(jnp.where(maskf_2d, mf.reshape(_H, _B), 0.0))
             + jnp.sum(jnp.where(maskb_2d, mb.reshape(_H, _B), 0.0)))

    i3 = jax.lax.broadcasted_iota(jnp.int32, (_H, _B, 2 * _K), 0)
    lane3 = jax.lax.broadcasted_iota(jnp.int32, (_H, _B, 2 * _K), 2)
    tval = jnp.where(lane3 < _K, i3, _T - 1 - i3)
    mask3 = tval < len_b2k.reshape(1, _B, 2 * _K)
    ecat_ref[...] = jnp.where(
        mask3, jnp.exp(emis - mbrd).reshape(_H, _B, 2 * _K), 1.0)

    lane2 = jax.lax.broadcasted_iota(jnp.int32, (_B, 2 * _K), 1)
    z0 = jnp.where(lane2 >= _K, 1.0,
                   jnp.where(lane2 == 0, 1.0, 0.0)).astype(jnp.float32)
    ones_b1 = jnp.ones((_B, 1), dtype=jnp.float32)
    zeros_b1 = jnp.zeros((_B, 1), dtype=jnp.float32)
    fmask = (lane2 < _K).astype(jnp.float32)
    Pbig_h = Pbig.astype(jnp.bfloat16)

    def body(i, carry):
        z, rf, rb, logzf, logzb = carry
        ec = ecat_ref[i]
        r2 = jnp.where(lane2 < _K, rf, rb)
        y = jnp.dot(z.astype(jnp.bfloat16), Pbig_h,
                    preferred_element_type=jnp.float32) * (ec * r2)
        sf = jnp.sum(y * fmask, axis=1, keepdims=True)
        sb = jnp.sum(y, axis=1, keepdims=True) - sf
        logzf = logzf + jnp.log(sf)
        logzb = logzb + jnp.log(sb)
        return y, 1.0 / sf, _K / sb, logzf, logzb

    z, _, _, logzf, logzb = jax.lax.fori_loop(
        0, _H, body, (z0, ones_b1, ones_b1, zeros_b1, zeros_b1),
        unroll=8)

    u = z[:, :_K]
    wz = z[:, _K:]
    w = jnp.dot(wz, PT, preferred_element_type=jnp.float32)
    su = jnp.sum(u, axis=1, keepdims=True)
    swz = jnp.sum(wz, axis=1, keepdims=True)
    comb = jnp.log(jnp.sum(u * w, axis=1, keepdims=True) / (su * swz))
    loglik = (jnp.sum(logzf + logzb + comb)
              + _B * (1.0 - _H) * math.log(_K) + m_sum)

    dir_const = _K * math.lgamma(1.0 + 0.1 * (_K - 1)) \
        - _K * (_K - 1) * math.lgamma(0.1)
    trace_lpx = jnp.sum(jnp.where(
        jax.lax.broadcasted_iota(jnp.int32, (_K, _K), 0)
        == jax.lax.broadcasted_iota(jnp.int32, (_K, _K), 1), log_px, 0.0))
    dir_lp = 0.9 * (trace_lpx - jnp.sum(log_px)) + dir_const
    beta_const = -_K * _D * (math.lgamma(0.1) + math.lgamma(0.9))
    beta_lp = -0.9 * jnp.sum(log_py) - 0.1 * jnp.sum(log_1mpy) + beta_const

    out_ref[0, 0] = loglik + dir_lp + beta_lp


def kernel(sequences, lengths, probs_x, probs_y):
    seq_t = jnp.swapaxes(sequences, 0, 1)
    seq_cat = jnp.concatenate(
        [seq_t[:_H], seq_t[::-1][:_H]], axis=-1)
    pxt = probs_x.T
    zP = jnp.zeros_like(probs_x)
    pbig = jnp.concatenate(
        [jnp.concatenate([probs_x, zP], axis=1),
         jnp.concatenate([zP, pxt], axis=1)], axis=0)
    len2d = jnp.broadcast_to(
        lengths.astype(jnp.int32).reshape(_B, 1), (_B, 2 * _K))
    out = pl.pallas_call(
        _hmm_kernel,
        out_shape=jax.ShapeDtypeStruct((1, 1), jnp.float32),
        out_specs=pl.BlockSpec(memory_space=pltpu.SMEM),
        scratch_shapes=[pltpu.VMEM((_H, _B, 2 * _K), jnp.float32)],
    )(seq_cat, len2d, pbig, pxt, probs_y)
    return out.reshape(())


# speedup vs baseline: 1.0064x; 1.0064x over previous
import math

import jax
import jax.numpy as jnp
from jax.experimental import pallas as pl
from jax.experimental.pallas import tpu as pltpu

_B, _T, _D, _K = 16, 512, 128, 64
_H = _T // 2


def _hmm_kernel(seqc_ref, len_ref, pbig_ref, pxt_ref, py_ref, out_ref,
                ef_ref, eb_ref):
    Pbig = pbig_ref[...]
    PT = pxt_ref[...]
    py = py_ref[...]
    log_py = jnp.log(py)
    log_1mpy = jnp.log1p(-py)
    log_px = jnp.log(Pbig[:_K, :_K])

    W = (log_py - log_1mpy).T
    zW = jnp.zeros_like(W)
    Wbig = jnp.concatenate(
        [jnp.concatenate([W, zW], axis=1),
         jnp.concatenate([zW, W], axis=1)], axis=0)
    bias = jnp.sum(log_1mpy, axis=1)[None, :]
    bias2 = jnp.concatenate([bias, bias], axis=1)
    seqc = seqc_ref[...].reshape(_H * _B, 2 * _D)
    emis = jnp.dot(seqc, Wbig, preferred_element_type=jnp.float32) + bias2

    lane = jax.lax.broadcasted_iota(jnp.int32, (_H * _B, 2 * _K), 1)
    is_f = lane < _K
    neg = jnp.float32(-1e30)
    mf = jnp.max(jnp.where(is_f, emis, neg), axis=1, keepdims=True)
    mb = jnp.max(jnp.where(is_f, neg, emis), axis=1, keepdims=True)
    mbrd = jnp.where(is_f, mf, mb)

    len_b2k = len_ref[...]
    i2 = jax.lax.broadcasted_iota(jnp.int32, (_H, _B), 0)
    len_row = len_b2k[:, :1].reshape(1, _B)
    maskf_2d = i2 < len_row
    maskb_2d = (_T - 1 - i2) < len_row
    m_sum = (jnp.sum(jnp.where(maskf_2d, mf.reshape(_H, _B), 0.0))
             + jnp.sum(jnp.where(maskb_2d, mb.reshape(_H, _B), 0.0)))

    i3 = jax.lax.broadcasted_iota(jnp.int32, (_H, _B, 2 * _K), 0)
    lane3 = jax.lax.broadcasted_iota(jnp.int32, (_H, _B, 2 * _K), 2)
    tval = jnp.where(lane3 < _K, i3, _T - 1 - i3)
    mask3 = tval < len_b2k.reshape(1, _B, 2 * _K)
    ecat = jnp.where(
        mask3, jnp.exp(emis - mbrd).reshape(_H, _B, 2 * _K), 1.0)
    ef_ref[...] = ecat[:, :, :_K]
    eb_ref[...] = ecat[:, :, _K:]

    P = Pbig[:_K, :_K]
    k_ids = jax.lax.broadcasted_iota(jnp.int32, (_B, _K), 1)
    u0 = jnp.where(k_ids == 0, 1.0, 0.0).astype(jnp.float32)
    w0 = jnp.ones((_B, _K), dtype=jnp.float32)
    ones_b1 = jnp.ones((_B, 1), dtype=jnp.float32)
    zeros_b1 = jnp.zeros((_B, 1), dtype=jnp.float32)

    def body(i, carry):
        u, w, rf, rb, logzf, logzb = carry
        ef = ef_ref[i]
        eb = eb_ref[i]
        v = jnp.dot(u, P, preferred_element_type=jnp.float32) * (ef * rf)
        x = jnp.dot(w, PT, preferred_element_type=jnp.float32) * (eb * rb)
        sf = jnp.sum(v, axis=1, keepdims=True)
        sb = jnp.sum(x, axis=1, keepdims=True)
        logzf = logzf + jnp.log(sf)
        logzb = logzb + jnp.log(sb)
        return v, x, 1.0 / sf, _K / sb, logzf, logzb

    u, wz, _, _, logzf, logzb = jax.lax.fori_loop(
        0, _H, body, (u0, w0, ones_b1, ones_b1, zeros_b1, zeros_b1),
        unroll=8)

    w = jnp.dot(wz, PT, preferred_element_type=jnp.float32)
    su = jnp.sum(u, axis=1, keepdims=True)
    swz = jnp.sum(wz, axis=1, keepdims=True)
    comb = jnp.log(jnp.sum(u * w, axis=1, keepdims=True) / (su * swz))
    loglik = (jnp.sum(logzf + logzb + comb)
              + _B * (1.0 - _H) * math.log(_K) + m_sum)

    dir_const = _K * math.lgamma(1.0 + 0.1 * (_K - 1)) \
        - _K * (_K - 1) * math.lgamma(0.1)
    trace_lpx = jnp.sum(jnp.where(
        jax.lax.broadcasted_iota(jnp.int32, (_K, _K), 0)
        == jax.lax.broadcasted_iota(jnp.int32, (_K, _K), 1), log_px, 0.0))
    dir_lp = 0.9 * (trace_lpx - jnp.sum(log_px)) + dir_const
    beta_const = -_K * _D * (math.lgamma(0.1) + math.lgamma(0.9))
    beta_lp = -0.9 * jnp.sum(log_py) - 0.1 * jnp.sum(log_1mpy) + beta_const

    out_ref[0, 0] = loglik + dir_lp + beta_lp


def kernel(sequences, lengths, probs_x, probs_y):
    seq_t = jnp.swapaxes(sequences, 0, 1)
    seq_cat = jnp.concatenate(
        [seq_t[:_H], seq_t[::-1][:_H]], axis=-1)
    pxt = probs_x.T
    zP = jnp.zeros_like(probs_x)
    pbig = jnp.concatenate(
        [jnp.concatenate([probs_x, zP], axis=1),
         jnp.concatenate([zP, pxt], axis=1)], axis=0)
    len2d = jnp.broadcast_to(
        lengths.astype(jnp.int32).reshape(_B, 1), (_B, 2 * _K))
    out = pl.pallas_call(
        _hmm_kernel,
        out_shape=jax.ShapeDtypeStruct((1, 1), jnp.float32),
        out_specs=pl.BlockSpec(memory_space=pltpu.SMEM),
        scratch_shapes=[pltpu.VMEM((_H, _B, _K), jnp.float32),
                        pltpu.VMEM((_H, _B, _K), jnp.float32)],
    )(seq_cat, len2d, pbig, pxt, probs_y)
    return out.reshape(())


# speedup vs baseline: 1.0451x; 1.0385x over previous
import math

import jax
import jax.numpy as jnp
from jax.experimental import pallas as pl
from jax.experimental.pallas import tpu as pltpu

_B, _T, _D, _K = 16, 512, 128, 64
_H = _T // 2


def _hmm_kernel(seqc_ref, len_ref, pbig_ref, pxt_ref, py_ref, out_ref,
                ecat_ref):
    Pbig = pbig_ref[...]
    PT = pxt_ref[...]
    py = py_ref[...]
    log_py = jnp.log(py)
    log_1mpy = jnp.log1p(-py)
    log_px = jnp.log(Pbig[:_K, :_K])

    W = (log_py - log_1mpy).T
    zW = jnp.zeros_like(W)
    Wbig = jnp.concatenate(
        [jnp.concatenate([W, zW], axis=1),
         jnp.concatenate([zW, W], axis=1)], axis=0)
    bias = jnp.sum(log_1mpy, axis=1)[None, :]
    bias2 = jnp.concatenate([bias, bias], axis=1)
    seqc = seqc_ref[...].reshape(_H * _B, 2 * _D)
    emis = jnp.dot(seqc, Wbig, preferred_element_type=jnp.float32) + bias2

    lane = jax.lax.broadcasted_iota(jnp.int32, (_H * _B, 2 * _K), 1)
    is_f = lane < _K
    neg = jnp.float32(-1e30)
    mf = jnp.max(jnp.where(is_f, emis, neg), axis=1, keepdims=True)
    mb = jnp.max(jnp.where(is_f, neg, emis), axis=1, keepdims=True)
    mbrd = jnp.where(is_f, mf, mb)

    len_b2k = len_ref[...]
    i2 = jax.lax.broadcasted_iota(jnp.int32, (_H, _B), 0)
    len_row = len_b2k[:, :1].reshape(1, _B)
    maskf_2d = i2 < len_row
    maskb_2d = (_T - 1 - i2) < len_row
    m_sum = (jnp.sum(jnp.where(maskf_2d, mf.reshape(_H, _B), 0.0))
             + jnp.sum(jnp.where(maskb_2d, mb.reshape(_H, _B), 0.0)))

    i3 = jax.lax.broadcasted_iota(jnp.int32, (_H, _B, 2 * _K), 0)
    lane3 = jax.lax.broadcasted_iota(jnp.int32, (_H, _B, 2 * _K), 2)
    tval = jnp.where(lane3 < _K, i3, _T - 1 - i3)
    mask3 = tval < len_b2k.reshape(1, _B, 2 * _K)
    ecat_ref[...] = jnp.where(
        mask3, jnp.exp(emis - mbrd).reshape(_H, _B, 2 * _K), 1.0)

    lane2 = jax.lax.broadcasted_iota(jnp.int32, (_B, 2 * _K), 1)
    z0 = jnp.where(lane2 >= _K, 1.0,
                   jnp.where(lane2 == 0, 1.0, 0.0)).astype(jnp.float32)
    ones_b1 = jnp.ones((_B, 1), dtype=jnp.float32)
    zeros_b1 = jnp.zeros((_B, 1), dtype=jnp.float32)
    fmask = (lane2 < _K).astype(jnp.float32)

    def body(i, carry):
        z, rf, rb, logzf, logzb = carry
        ec = ecat_ref[i]
        r2 = jnp.where(lane2 < _K, rf, rb)
        y = jnp.dot(z, Pbig, preferred_element_type=jnp.float32) * (ec * r2)
        sf = jnp.sum(y * fmask, axis=1, keepdims=True)
        sb = jnp.sum(y, axis=1, keepdims=True) - sf
        logzf = logzf + jnp.log(sf)
        logzb = logzb + jnp.log(sb)
        return y, 1.0 / sf, _K / sb, logzf, logzb

    z, _, _, logzf, logzb = jax.lax.fori_loop(
        0, _H, body, (z0, ones_b1, ones_b1, zeros_b1, zeros_b1),
        unroll=16)

    u = z[:, :_K]
    wz = z[:, _K:]
    w = jnp.dot(wz, PT, preferred_element_type=jnp.float32)
    su = jnp.sum(u, axis=1, keepdims=True)
    swz = jnp.sum(wz, axis=1, keepdims=True)
    comb = jnp.log(jnp.sum(u * w, axis=1, keepdims=True) / (su * swz))
    loglik = (jnp.sum(logzf + logzb + comb)
              + _B * (1.0 - _H) * math.log(_K) + m_sum)

    dir_const = _K * math.lgamma(1.0 + 0.1 * (_K - 1)) \
        - _K * (_K - 1) * math.lgamma(0.1)
    trace_lpx = jnp.sum(jnp.where(
        jax.lax.broadcasted_iota(jnp.int32, (_K, _K), 0)
        == jax.lax.broadcasted_iota(jnp.int32, (_K, _K), 1), log_px, 0.0))
    dir_lp = 0.9 * (trace_lpx - jnp.sum(log_px)) + dir_const
    beta_const = -_K * _D * (math.lgamma(0.1) + math.lgamma(0.9))
    beta_lp = -0.9 * jnp.sum(log_py) - 0.1 * jnp.sum(log_1mpy) + beta_const

    out_ref[0, 0] = loglik + dir_lp + beta_lp


def kernel(sequences, lengths, probs_x, probs_y):
    seq_t = jnp.swapaxes(sequences, 0, 1)
    seq_cat = jnp.concatenate(
        [seq_t[:_H], seq_t[::-1][:_H]], axis=-1)
    pxt = probs_x.T
    zP = jnp.zeros_like(probs_x)
    pbig = jnp.concatenate(
        [jnp.concatenate([probs_x, zP], axis=1),
         jnp.concatenate([zP, pxt], axis=1)], axis=0)
    len2d = jnp.broadcast_to(
        lengths.astype(jnp.int32).reshape(_B, 1), (_B, 2 * _K))
    out = pl.pallas_call(
        _hmm_kernel,
        out_shape=jax.ShapeDtypeStruct((1, 1), jnp.float32),
        out_specs=pl.BlockSpec(memory_space=pltpu.SMEM),
        scratch_shapes=[pltpu.VMEM((_H, _B, 2 * _K), jnp.float32)],
    )(seq_cat, len2d, pbig, pxt, probs_y)
    return out.reshape(())


# speedup vs baseline: 1.0654x; 1.0194x over previous
import math

import jax
import jax.numpy as jnp
from jax.experimental import pallas as pl
from jax.experimental.pallas import tpu as pltpu

_B, _T, _D, _K = 16, 512, 128, 64
_H = _T // 2


def _hmm_kernel(seqc_ref, len_ref, pbig_ref, pxt_ref, py_ref, out_ref,
                ecat_ref):
    Pbig = pbig_ref[...]
    PT = pxt_ref[...]
    py = py_ref[...]
    log_py = jnp.log(py)
    log_1mpy = jnp.log1p(-py)
    log_px = jnp.log(Pbig[:_K, :_K])

    W = (log_py - log_1mpy).T
    zW = jnp.zeros_like(W)
    Wbig = jnp.concatenate(
        [jnp.concatenate([W, zW], axis=1),
         jnp.concatenate([zW, W], axis=1)], axis=0)
    bias = jnp.sum(log_1mpy, axis=1)[None, :]
    bias2 = jnp.concatenate([bias, bias], axis=1)
    seqc = seqc_ref[...].reshape(_H * _B, 2 * _D)
    emis = jnp.dot(seqc, Wbig, preferred_element_type=jnp.float32) + bias2

    lane = jax.lax.broadcasted_iota(jnp.int32, (_H * _B, 2 * _K), 1)
    is_f = lane < _K
    neg = jnp.float32(-1e30)
    mf = jnp.max(jnp.where(is_f, emis, neg), axis=1, keepdims=True)
    mb = jnp.max(jnp.where(is_f, neg, emis), axis=1, keepdims=True)
    mbrd = jnp.where(is_f, mf, mb)

    len_b2k = len_ref[...]
    i2 = jax.lax.broadcasted_iota(jnp.int32, (_H, _B), 0)
    len_row = len_b2k[:, :1].reshape(1, _B)
    maskf_2d = i2 < len_row
    maskb_2d = (_T - 1 - i2) < len_row
    m_sum = (jnp.sum(jnp.where(maskf_2d, mf.reshape(_H, _B), 0.0))
             + jnp.sum(jnp.where(maskb_2d, mb.reshape(_H, _B), 0.0)))

    i3 = jax.lax.broadcasted_iota(jnp.int32, (_H, _B, 2 * _K), 0)
    lane3 = jax.lax.broadcasted_iota(jnp.int32, (_H, _B, 2 * _K), 2)
    tval = jnp.where(lane3 < _K, i3, _T - 1 - i3)
    mask3 = tval < len_b2k.reshape(1, _B, 2 * _K)
    ecat_ref[...] = jnp.where(
        mask3, jnp.exp(emis - mbrd).reshape(_H, _B, 2 * _K), 1.0)

    lane2 = jax.lax.broadcasted_iota(jnp.int32, (_B, 2 * _K), 1)
    z0 = jnp.where(lane2 >= _K, 1.0,
                   jnp.where(lane2 == 0, 1.0, 0.0)).astype(jnp.float32)
    ones_b1 = jnp.ones((_B, 1), dtype=jnp.float32)
    zeros_b1 = jnp.zeros((_B, 1), dtype=jnp.float32)
    fmask = (lane2 < _K).astype(jnp.float32)

    def body(i, carry):
        z, rf, rb, logzf, logzb = carry
        ec = ecat_ref[i]
        r2 = jnp.where(lane2 < _K, rf, rb)
        y = jnp.dot(z, Pbig, preferred_element_type=jnp.float32) * (ec * r2)
        sf = jnp.sum(y * fmask, axis=1, keepdims=True)
        sb = jnp.sum(y, axis=1, keepdims=True) - sf
        logzf = logzf + jnp.log(sf)
        logzb = logzb + jnp.log(sb)
        return y, 1.0 / sf, _K / sb, logzf, logzb

    z, _, _, logzf, logzb = jax.lax.fori_loop(
        0, _H, body, (z0, ones_b1, ones_b1, zeros_b1, zeros_b1),
        unroll=32)

    u = z[:, :_K]
    wz = z[:, _K:]
    w = jnp.dot(wz, PT, preferred_element_type=jnp.float32)
    su = jnp.sum(u, axis=1, keepdims=True)
    swz = jnp.sum(wz, axis=1, keepdims=True)
    comb = jnp.log(jnp.sum(u * w, axis=1, keepdims=True) / (su * swz))
    loglik = (jnp.sum(logzf + logzb + comb)
              + _B * (1.0 - _H) * math.log(_K) + m_sum)

    dir_const = _K * math.lgamma(1.0 + 0.1 * (_K - 1)) \
        - _K * (_K - 1) * math.lgamma(0.1)
    trace_lpx = jnp.sum(jnp.where(
        jax.lax.broadcasted_iota(jnp.int32, (_K, _K), 0)
        == jax.lax.broadcasted_iota(jnp.int32, (_K, _K), 1), log_px, 0.0))
    dir_lp = 0.9 * (trace_lpx - jnp.sum(log_px)) + dir_const
    beta_const = -_K * _D * (math.lgamma(0.1) + math.lgamma(0.9))
    beta_lp = -0.9 * jnp.sum(log_py) - 0.1 * jnp.sum(log_1mpy) + beta_const

    out_ref[0, 0] = loglik + dir_lp + beta_lp


def kernel(sequences, lengths, probs_x, probs_y):
    seq_t = jnp.swapaxes(sequences, 0, 1)
    seq_cat = jnp.concatenate(
        [seq_t[:_H], seq_t[::-1][:_H]], axis=-1)
    pxt = probs_x.T
    zP = jnp.zeros_like(probs_x)
    pbig = jnp.concatenate(
        [jnp.concatenate([probs_x, zP], axis=1),
         jnp.concatenate([zP, pxt], axis=1)], axis=0)
    len2d = jnp.broadcast_to(
        lengths.astype(jnp.int32).reshape(_B, 1), (_B, 2 * _K))
    out = pl.pallas_call(
        _hmm_kernel,
        out_shape=jax.ShapeDtypeStruct((1, 1), jnp.float32),
        out_specs=pl.BlockSpec(memory_space=pltpu.SMEM),
        scratch_shapes=[pltpu.VMEM((_H, _B, 2 * _K), jnp.float32)],
    )(seq_cat, len2d, pbig, pxt, probs_y)
    return out.reshape(())


# speedup vs baseline: 1.2216x; 1.1466x over previous
import math

import jax
import jax.numpy as jnp
from jax.experimental import pallas as pl
from jax.experimental.pallas import tpu as pltpu

_B, _T, _D, _K = 16, 512, 128, 64
_H = _T // 2


def _hmm_kernel(seq_ref, lenrow_ref, pbig_ref, pxt_ref, py_ref, out_ref,
                ecat_ref):
    Pbig = pbig_ref[...]
    PT = pxt_ref[...]
    py = py_ref[...]
    log_py = jnp.log(py)
    log_1mpy = jnp.log1p(-py)
    log_px = jnp.log(Pbig[:_K, :_K])

    W = (log_py - log_1mpy).T
    bias = jnp.sum(log_1mpy, axis=1)[None, :]
    seq = seq_ref[...].reshape(_B * _T, _D)
    emis = jnp.dot(seq, W, preferred_element_type=jnp.float32) + bias

    m = jnp.max(emis, axis=1, keepdims=True)
    lenrow = lenrow_ref[...]
    t_row1 = jax.lax.broadcasted_iota(jnp.int32, (_B * _T, 1), 0) & (_T - 1)
    m_sum = jnp.sum(jnp.where(t_row1 < lenrow, m, 0.0))

    t_rowk = jax.lax.broadcasted_iota(jnp.int32, (_B * _T, _K), 0) & (_T - 1)
    ecat_ref[...] = jnp.where(
        t_rowk < lenrow, jnp.exp(emis - m), 1.0).reshape(_B, _T, _K)

    lane2 = jax.lax.broadcasted_iota(jnp.int32, (_B, 2 * _K), 1)
    z0 = jnp.where(lane2 >= _K, 1.0,
                   jnp.where(lane2 == 0, 1.0, 0.0)).astype(jnp.float32)
    ones_b1 = jnp.ones((_B, 1), dtype=jnp.float32)
    zeros_b1 = jnp.zeros((_B, 1), dtype=jnp.float32)
    fmask = (lane2 < _K).astype(jnp.float32)

    def body(i, carry):
        z, rf, rb, logzf, logzb = carry
        ef = ecat_ref[:, i]
        eb = ecat_ref[:, _T - 1 - i]
        ec = jnp.concatenate([ef, eb], axis=1)
        r2 = jnp.where(lane2 < _K, rf, rb)
        y = jnp.dot(z, Pbig, preferred_element_type=jnp.float32) * (ec * r2)
        sf = jnp.sum(y * fmask, axis=1, keepdims=True)
        sb = jnp.sum(y, axis=1, keepdims=True) - sf
        logzf = logzf + jnp.log(sf)
        logzb = logzb + jnp.log(sb)
        return y, 1.0 / sf, _K / sb, logzf, logzb

    z, _, _, logzf, logzb = jax.lax.fori_loop(
        0, _H, body, (z0, ones_b1, ones_b1, zeros_b1, zeros_b1),
        unroll=32)

    u = z[:, :_K]
    wz = z[:, _K:]
    w = jnp.dot(wz, PT, preferred_element_type=jnp.float32)
    su = jnp.sum(u, axis=1, keepdims=True)
    swz = jnp.sum(wz, axis=1, keepdims=True)
    comb = jnp.log(jnp.sum(u * w, axis=1, keepdims=True) / (su * swz))
    loglik = (jnp.sum(logzf + logzb + comb)
              + _B * (1.0 - _H) * math.log(_K) + m_sum)

    dir_const = _K * math.lgamma(1.0 + 0.1 * (_K - 1)) \
        - _K * (_K - 1) * math.lgamma(0.1)
    trace_lpx = jnp.sum(jnp.where(
        jax.lax.broadcasted_iota(jnp.int32, (_K, _K), 0)
        == jax.lax.broadcasted_iota(jnp.int32, (_K, _K), 1), log_px, 0.0))
    dir_lp = 0.9 * (trace_lpx - jnp.sum(log_px)) + dir_const
    beta_const = -_K * _D * (math.lgamma(0.1) + math.lgamma(0.9))
    beta_lp = -0.9 * jnp.sum(log_py) - 0.1 * jnp.sum(log_1mpy) + beta_const

    out_ref[0, 0] = loglik + dir_lp + beta_lp


def kernel(sequences, lengths, probs_x, probs_y):
    pxt = probs_x.T
    zP = jnp.zeros_like(probs_x)
    pbig = jnp.concatenate(
        [jnp.concatenate([probs_x, zP], axis=1),
         jnp.concatenate([zP, pxt], axis=1)], axis=0)
    lenrow = jnp.broadcast_to(
        lengths.astype(jnp.int32).reshape(_B, 1), (_B, _T)).reshape(
            _B * _T, 1)
    out = pl.pallas_call(
        _hmm_kernel,
        out_shape=jax.ShapeDtypeStruct((1, 1), jnp.float32),
        out_specs=pl.BlockSpec(memory_space=pltpu.SMEM),
        scratch_shapes=[pltpu.VMEM((_B, _T, _K), jnp.float32)],
    )(sequences, lenrow, pbig, pxt, probs_y)
    return out.reshape(())


# speedup vs baseline: 1.2338x; 1.0100x over previous
import math

import jax
import jax.numpy as jnp
from jax.experimental import pallas as pl
from jax.experimental.pallas import tpu as pltpu

_B, _T, _D, _K = 16, 512, 128, 64
_H = _T // 2


def _hmm_kernel(seq_ref, lenrow_ref, pbig_ref, pxt_ref, py_ref, out_ref,
                ecat_ref):
    Pbig = pbig_ref[...]
    PT = pxt_ref[...]
    py = py_ref[...]
    log_py = jnp.log(py)
    log_1mpy = jnp.log1p(-py)
    log_px = jnp.log(Pbig[:_K, :_K])

    W = (log_py - log_1mpy).T
    bias = jnp.sum(log_1mpy, axis=1)[None, :]
    seq = seq_ref[...].reshape(_B * _T, _D)
    emis = jnp.dot(seq, W, preferred_element_type=jnp.float32) + bias

    m = jnp.max(emis, axis=1, keepdims=True)
    lenrow = lenrow_ref[...]
    t_row1 = jax.lax.broadcasted_iota(jnp.int32, (_B * _T, 1), 0) & (_T - 1)
    m_sum = jnp.sum(jnp.where(t_row1 < lenrow, m, 0.0))

    t_rowk = jax.lax.broadcasted_iota(jnp.int32, (_B * _T, _K), 0) & (_T - 1)
    ecat_ref[...] = jnp.where(
        t_rowk < lenrow, jnp.exp(emis - m), 1.0).reshape(_B, _T, _K)

    lane2 = jax.lax.broadcasted_iota(jnp.int32, (_B, 2 * _K), 1)
    z0 = jnp.where(lane2 >= _K, 1.0,
                   jnp.where(lane2 == 0, 1.0, 0.0)).astype(jnp.float32)
    ones_b1 = jnp.ones((_B, 1), dtype=jnp.float32)
    zeros_b1 = jnp.zeros((_B, 1), dtype=jnp.float32)
    fmask = (lane2 < _K).astype(jnp.float32)

    def body(i, carry):
        z, rf, rb, logzf, logzb = carry
        ef = ecat_ref[:, i]
        eb = ecat_ref[:, _T - 1 - i]
        ec = jnp.concatenate([ef, eb], axis=1)
        r2 = jnp.where(lane2 < _K, rf, rb)
        y = jnp.dot(z, Pbig, preferred_element_type=jnp.float32) * (ec * r2)
        sf = jnp.sum(y * fmask, axis=1, keepdims=True)
        sb = jnp.sum(y, axis=1, keepdims=True) - sf
        logzf = logzf + jnp.log(sf)
        logzb = logzb + jnp.log(sb)
        return y, 1.0 / sf, _K / sb, logzf, logzb

    z, _, _, logzf, logzb = jax.lax.fori_loop(
        0, _H, body, (z0, ones_b1, ones_b1, zeros_b1, zeros_b1),
        unroll=64)

    u = z[:, :_K]
    wz = z[:, _K:]
    w = jnp.dot(wz, PT, preferred_element_type=jnp.float32)
    su = jnp.sum(u, axis=1, keepdims=True)
    swz = jnp.sum(wz, axis=1, keepdims=True)
    comb = jnp.log(jnp.sum(u * w, axis=1, keepdims=True) / (su * swz))
    loglik = (jnp.sum(logzf + logzb + comb)
              + _B * (1.0 - _H) * math.log(_K) + m_sum)

    dir_const = _K * math.lgamma(1.0 + 0.1 * (_K - 1)) \
        - _K * (_K - 1) * math.lgamma(0.1)
    trace_lpx = jnp.sum(jnp.where(
        jax.lax.broadcasted_iota(jnp.int32, (_K, _K), 0)
        == jax.lax.broadcasted_iota(jnp.int32, (_K, _K), 1), log_px, 0.0))
    dir_lp = 0.9 * (trace_lpx - jnp.sum(log_px)) + dir_const
    beta_const = -_K * _D * (math.lgamma(0.1) + math.lgamma(0.9))
    beta_lp = -0.9 * jnp.sum(log_py) - 0.1 * jnp.sum(log_1mpy) + beta_const

    out_ref[0, 0] = loglik + dir_lp + beta_lp


def kernel(sequences, lengths, probs_x, probs_y):
    pxt = probs_x.T
    zP = jnp.zeros_like(probs_x)
    pbig = jnp.concatenate(
        [jnp.concatenate([probs_x, zP], axis=1),
         jnp.concatenate([zP, pxt], axis=1)], axis=0)
    lenrow = jnp.broadcast_to(
        lengths.astype(jnp.int32).reshape(_B, 1), (_B, _T)).reshape(
            _B * _T, 1)
    out = pl.pallas_call(
        _hmm_kernel,
        out_shape=jax.ShapeDtypeStruct((1, 1), jnp.float32),
        out_specs=pl.BlockSpec(memory_space=pltpu.SMEM),
        scratch_shapes=[pltpu.VMEM((_B, _T, _K), jnp.float32)],
    )(sequences, lenrow, pbig, pxt, probs_y)
    return out.reshape(())
